# Initial kernel scaffold; baseline (speedup 1.0000x reference)
import jax
import jax.numpy as jnp
from jax.experimental import pallas as pl


def kernel(num_feats, cat_feats, W_num, b_num, tables, ctr_W1, ctr_b1, ctr_W2, ctr_b2, cvr_W1, cvr_b1, cvr_W2, cvr_b2):
    raise NotImplementedError("write your pallas kernel here")


# speedup vs baseline: 7.7529x; 7.7529x over previous
import functools

import jax
import jax.numpy as jnp
from jax import lax
from jax.experimental import pallas as pl
from jax.experimental.pallas import tpu as pltpu
from jax.experimental.pallas import tpu_sc as plsc

B = 16384
NUM_DIM = 13
N_FIELDS = 26
VOCAB = 100000
EMB = 16
MLP = 32
TASK = 512

ROWS = B * N_FIELDS
NC, NS = 2, 16
NW = NC * NS
IDXW = 128
IDX_ROWS = ROWS // IDXW
IDX_ROWS_PER_W = IDX_ROWS // NW
K = 13
NCH = IDX_ROWS_PER_W // K
RPC = K * IDXW

BB = 2048


def _sc_gather(flat_idx, flat_table):
    mesh = plsc.VectorSubcoreMesh(core_axis_name="c", subcore_axis_name="s")

    @functools.partial(
        pl.kernel,
        mesh=mesh,
        out_type=jax.ShapeDtypeStruct((IDX_ROWS, IDXW, EMB), jnp.float32),
        scratch_types=[
            pltpu.VMEM((IDX_ROWS_PER_W, IDXW), jnp.int32),
            pltpu.VMEM((K, IDXW, EMB), jnp.float32),
            pltpu.SemaphoreType.DMA,
        ],
        compiler_params=pltpu.CompilerParams(use_tc_tiling_on_sc=False),
    )
    def gather_kernel(idx_hbm, table_hbm, out_hbm, idx_v, rows_v, gsem):
        wid = lax.axis_index("s") * NC + lax.axis_index("c")
        ibase = wid * IDX_ROWS_PER_W
        pltpu.sync_copy(idx_hbm.at[pl.ds(ibase, IDX_ROWS_PER_W)], idx_v)

        def chunk(j, carry):
            base = j * K
            cps = []
            for t in range(K):
                cps.append(pltpu.async_copy(
                    table_hbm.at[idx_v.at[base + t]],
                    rows_v.at[t], gsem))
            for cp in cps:
                cp.wait()
            pltpu.sync_copy(rows_v, out_hbm.at[pl.ds(ibase + base, K)])
            return carry

        lax.fori_loop(0, NCH, chunk, 0)

    return gather_kernel(flat_idx, flat_table)


def _tc_dense(num_feats, cat_out, W_num, b_num,
              ctr_W1, ctr_b1, ctr_w2, ctr_b2,
              cvr_W1, cvr_b1, cvr_w2, cvr_b2):
    def body(nf, cat, Wn, bn, cW1, cb1, cw2, cb2, vW1, vb1, vw2, vb2,
             octr, ocvr):
        num_out = jnp.dot(nf[...], Wn[...],
                          preferred_element_type=jnp.float32) + bn[...]
        shared = jnp.concatenate([num_out, cat[...]], axis=1)
        hc = jnp.maximum(
            jnp.dot(shared, cW1[...], preferred_element_type=jnp.float32)
            + cb1[...], 0.0)
        octr[...] = jnp.sum(hc * cw2[...], axis=1, keepdims=True) + cb2[...]
        hv = jnp.maximum(
            jnp.dot(shared, vW1[...], preferred_element_type=jnp.float32)
            + vb1[...], 0.0)
        ocvr[...] = jnp.sum(hv * vw2[...], axis=1, keepdims=True) + vb2[...]

    full = lambda shape: pl.BlockSpec(shape, lambda i: (0, 0))
    grid = (B // BB,)
    return pl.pallas_call(
        body,
        grid=grid,
        in_specs=[
            pl.BlockSpec((BB, NUM_DIM), lambda i: (i, 0)),
            pl.BlockSpec((BB, N_FIELDS * EMB), lambda i: (i, 0)),
            full((NUM_DIM, MLP)),
            full((1, MLP)),
            full((MLP + N_FIELDS * EMB, TASK)),
            full((1, TASK)),
            full((1, TASK)),
            full((1, 1)),
            full((MLP + N_FIELDS * EMB, TASK)),
            full((1, TASK)),
            full((1, TASK)),
            full((1, 1)),
        ],
        out_specs=[
            pl.BlockSpec((BB, 1), lambda i: (i, 0)),
            pl.BlockSpec((BB, 1), lambda i: (i, 0)),
        ],
        out_shape=[
            jax.ShapeDtypeStruct((B, 1), jnp.float32),
            jax.ShapeDtypeStruct((B, 1), jnp.float32),
        ],
    )(num_feats, cat_out, W_num, b_num,
      ctr_W1, ctr_b1, ctr_w2, ctr_b2,
      cvr_W1, cvr_b1, cvr_w2, cvr_b2)


def kernel(num_feats, cat_feats, W_num, b_num, tables,
           ctr_W1, ctr_b1, ctr_W2, ctr_b2,
           cvr_W1, cvr_b1, cvr_W2, cvr_b2):
    flat_table = tables.reshape(N_FIELDS * VOCAB, EMB)
    offs = (jnp.arange(N_FIELDS, dtype=jnp.int32) * VOCAB)[None, :]
    flat_idx = (cat_feats.astype(jnp.int32) + offs).reshape(IDX_ROWS, IDXW)

    gathered = _sc_gather(flat_idx, flat_table)
    cat_out = gathered.reshape(B, N_FIELDS * EMB)

    ctr, cvr = _tc_dense(
        num_feats, cat_out, W_num, b_num.reshape(1, MLP),
        ctr_W1, ctr_b1.reshape(1, TASK), ctr_W2.reshape(1, TASK),
        ctr_b2.reshape(1, 1),
        cvr_W1, cvr_b1.reshape(1, TASK), cvr_W2.reshape(1, TASK),
        cvr_b2.reshape(1, 1))
    return (ctr, cvr)
